# baseline (speedup 1.0000x reference)
import functools

import jax
import jax.numpy as jnp
from jax import lax
from jax.experimental import pallas as pl
from jax.experimental.pallas import tpu as pltpu
from jax.experimental.pallas import tpu_sc as plsc

N = 10000
F = 512
H1 = 256
H2 = 128
E = 320000
CH = 128
DC = 128
E_PAD = 327680
NROW = E_PAD // CH


def _make_spmm(edge_split):
    n_tiles = 32 if edge_split else 16
    nrt = NROW // n_tiles
    BLK = 4
    nblocks = nrt // BLK
    stripe = N // 16

    mesh = plsc.VectorSubcoreMesh(core_axis_name="c", subcore_axis_name="s")

    @functools.partial(
        pl.kernel,
        out_type=jax.ShapeDtypeStruct((2 * N, DC), jnp.float32),
        mesh=mesh,
        scratch_types=[
            pltpu.VMEM((2, BLK, CH), jnp.int32),
            pltpu.VMEM((2, BLK, CH), jnp.int32),
            pltpu.VMEM((2, BLK, CH), jnp.float32),
            pltpu.VMEM((2, CH, DC), jnp.float32),
            pltpu.VMEM_SHARED((N, DC), jnp.float32),
            pltpu.SemaphoreType.DMA,
            pltpu.SemaphoreType.DMA,
            pltpu.SemaphoreType.DMA,
        ],
    )
    def spmm(cols3_hbm, rows2_hbm, vals2_hbm, dense_hbm, out_hbm,
             colsm, rowsm, valsm, gath, acc, semg, sems, semm):
        c = lax.axis_index("c")
        s = lax.axis_index("s")
        tile = c * 16 + s if edge_split else s
        row0 = tile * nrt
        cplane = 0 if edge_split else c

        def issue_meta(m, p):
            r = row0 + m * BLK
            pltpu.async_copy(cols3_hbm.at[cplane, pl.ds(r, BLK)],
                             colsm.at[p], semm)
            pltpu.async_copy(rows2_hbm.at[pl.ds(r, BLK)], rowsm.at[p], semm)
            pltpu.async_copy(vals2_hbm.at[pl.ds(r, BLK)], valsm.at[p], semm)

        def wait_meta(m, p):
            r = row0 + m * BLK
            pltpu.make_async_copy(cols3_hbm.at[cplane, pl.ds(r, BLK)],
                                  colsm.at[p], semm).wait()
            pltpu.make_async_copy(rows2_hbm.at[pl.ds(r, BLK)],
                                  rowsm.at[p], semm).wait()
            pltpu.make_async_copy(vals2_hbm.at[pl.ds(r, BLK)],
                                  valsm.at[p], semm).wait()

        def issue_gather(p, r, b):
            pltpu.async_copy(dense_hbm.at[colsm.at[p, r]], gath.at[b], semg)

        def wait_gather(p, r, b):
            pltpu.make_async_copy(dense_hbm.at[colsm.at[p, r]],
                                  gath.at[b], semg).wait()

        def issue_scatter(p, r, b):
            pltpu.async_copy(gath.at[b], acc.at[rowsm.at[p, r]], sems,
                             add=True)

        def wait_scatter(p, r, b):
            pltpu.make_async_copy(gath.at[b], acc.at[rowsm.at[p, r]],
                                  sems).wait()

        def scale(p, r, b):
            gbuf = gath.at[b]

            def tloop(t, tc):
                vals16 = valsm[p, r, pl.ds(t * 16, 16)]
                for e in range(16):
                    v = vals16[e]
                    row = t * 16 + e
                    for j in range(DC // 16):
                        sl = pl.ds(j * 16, 16)
                        gbuf[row, sl] = gbuf[row, sl] * v
                return tc

            lax.fori_loop(0, CH // 16, tloop, 0)

        issue_meta(0, 0)

        zero16 = jnp.zeros((16,), jnp.float32)
        zbuf = gath.at[0]

        def zrow(i, carry):
            for j in range(DC // 16):
                zbuf[i, pl.ds(j * 16, 16)] = zero16
            return carry

        lax.fori_loop(0, CH, zrow, 0)
        for q in range(stripe // 125):
            pltpu.sync_copy(zbuf.at[pl.ds(0, 125)],
                            acc.at[pl.ds(s * stripe + q * 125, 125)])
        wait_meta(0, 0)
        plsc.subcore_barrier()
        issue_gather(0, 0, 0)

        def body(mm, carry):
            for mb in range(2):
                P = mb
                m = mm * 2 + mb

                @pl.when(m + 1 < nblocks)
                def _():
                    issue_meta(m + 1, 1 - P)

                for b8 in range(BLK):
                    k = m * BLK + b8
                    b = b8 % 2
                    q = 1 - b
                    pq, rq = (1 - P, BLK - 1) if b8 == 0 else (P, b8 - 1)

                    @pl.when(k >= 1)
                    def _():
                        wait_scatter(pq, rq, q)

                    if b8 == BLK - 1:
                        @pl.when(m + 1 < nblocks)
                        def _():
                            wait_meta(m + 1, 1 - P)
                            issue_gather(1 - P, 0, q)
                    else:
                        issue_gather(P, b8 + 1, q)

                    wait_gather(P, b8, b)
                    scale(P, b8, b)
                    issue_scatter(P, b8, b)
            return carry

        lax.fori_loop(0, nblocks // 2, body, 0)
        wait_scatter(1, BLK - 1, (BLK - 1) % 2)
        plsc.subcore_barrier()

        w0 = s * 632

        @pl.when(s < 15)
        def _():
            pltpu.sync_copy(acc.at[pl.ds(w0, 632)],
                            out_hbm.at[pl.ds(c * N + w0, 632)])

        @pl.when(s == 15)
        def _():
            pltpu.sync_copy(acc.at[pl.ds(15 * 632, N - 15 * 632)],
                            out_hbm.at[pl.ds(c * N + 15 * 632, N - 15 * 632)])

    return spmm


_spmm_dsplit = _make_spmm(edge_split=False)
_spmm_esplit = _make_spmm(edge_split=True)


def _relu_w2_body(h0_ref, h1_ref, w_ref, o_ref):
    h0 = jnp.maximum(h0_ref[...], 0.0)
    h1 = jnp.maximum(h1_ref[...], 0.0)
    w = w_ref[...]
    o_ref[...] = (
        lax.dot_general(h0, w[:DC], (((1,), (0,)), ((), ())),
                        preferred_element_type=jnp.float32)
        + lax.dot_general(h1, w[DC:], (((1,), (0,)), ((), ())),
                          preferred_element_type=jnp.float32))


def _relu_w2(h_2, w2):
    bm = 2000
    return pl.pallas_call(
        _relu_w2_body,
        grid=(N // bm,),
        in_specs=[
            pl.BlockSpec((bm, DC), lambda i: (i, 0)),
            pl.BlockSpec((bm, DC), lambda i: (i + N // bm, 0)),
            pl.BlockSpec((H1, H2), lambda i: (0, 0)),
        ],
        out_specs=pl.BlockSpec((bm, H2), lambda i: (i, 0)),
        out_shape=jax.ShapeDtypeStruct((N, H2), jnp.float32),
    )(h_2, h_2, w2)


def _gram_body(a0_ref, a1_ref, b0_ref, b1_ref, o_ref):
    a = a0_ref[...] + a1_ref[...]
    b = b0_ref[...] + b1_ref[...]
    o_ref[...] = lax.dot_general(a, b, (((1,), (1,)), ((), ())),
                                 preferred_element_type=jnp.float32)


def _gram(p_2):
    bm = 200
    g = N // bm
    return pl.pallas_call(
        _gram_body,
        grid=(g,),
        in_specs=[
            pl.BlockSpec((bm, H2), lambda i: (i, 0)),
            pl.BlockSpec((bm, H2), lambda i: (i + g, 0)),
            pl.BlockSpec((N, H2), lambda i: (0, 0)),
            pl.BlockSpec((N, H2), lambda i: (1, 0)),
        ],
        out_specs=pl.BlockSpec((bm, N), lambda i: (i, 0)),
        out_shape=jax.ShapeDtypeStruct((N, N), jnp.float32),
    )(p_2, p_2, p_2, p_2)


def _prep_edges(rows, cols, vals, k_dim):
    pad = E_PAD - E
    z = jnp.zeros((pad,), jnp.int32)
    rows2 = jnp.concatenate([rows, z]).reshape(NROW, CH)
    cols_p = jnp.concatenate([cols, z]).reshape(NROW, CH)
    vals2 = jnp.concatenate(
        [vals, jnp.zeros((pad,), jnp.float32)]).reshape(NROW, CH)
    cols3 = jnp.stack([cols_p, cols_p + k_dim])
    return cols3, rows2, vals2


@jax.jit
def _run(feat_rows, feat_cols, feat_vals, adj_rows, adj_cols, adj_vals, W1, W2):
    fc3, fr2, fv2 = _prep_edges(feat_rows, feat_cols, feat_vals, F)
    ac3, ar2, av2 = _prep_edges(adj_rows, adj_cols, adj_vals, N)
    w1_2 = jnp.concatenate([W1[:, :DC], W1[:, DC:]], axis=0)
    h1_2 = _spmm_dsplit(fc3, fr2, fv2, w1_2)
    h_2 = _spmm_dsplit(ac3, ar2, av2, h1_2)
    h2 = _relu_w2(h_2, W2)
    p_2 = _spmm_esplit(ac3, ar2, av2, h2)
    recon = _gram(p_2)
    return recon.astype(jnp.float64)


def kernel(feat_rows, feat_cols, feat_vals, adj_rows, adj_cols, adj_vals, W1, W2):
    return _run(feat_rows, feat_cols, feat_vals, adj_rows, adj_cols, adj_vals, W1, W2)


# speedup vs baseline: 2.4414x; 2.4414x over previous
import functools

import jax
import jax.numpy as jnp
from jax import lax
from jax.experimental import pallas as pl
from jax.experimental.pallas import tpu as pltpu
from jax.experimental.pallas import tpu_sc as plsc

N = 10000
F = 512
H1 = 256
H2 = 128
E = 320000
CH = 128
DC = 128
E_PAD = 327680
NROW = E_PAD // CH


def _make_spmm(edge_split):
    n_tiles = 32 if edge_split else 16
    nrt = NROW // n_tiles
    BLK = 4
    nblocks = nrt // BLK
    stripe = N // 16

    mesh = plsc.VectorSubcoreMesh(core_axis_name="c", subcore_axis_name="s")

    @functools.partial(
        pl.kernel,
        out_type=jax.ShapeDtypeStruct((2 * N, DC), jnp.float32),
        mesh=mesh,
        scratch_types=[
            pltpu.VMEM((2, BLK, CH), jnp.int32),
            pltpu.VMEM((2, BLK, CH), jnp.int32),
            pltpu.VMEM((2, BLK, CH), jnp.float32),
            pltpu.VMEM((2, CH, DC), jnp.float32),
            pltpu.VMEM_SHARED((N, DC), jnp.float32),
            pltpu.SemaphoreType.DMA,
            pltpu.SemaphoreType.DMA,
            pltpu.SemaphoreType.DMA,
        ],
    )
    def spmm(cols3_hbm, rows2_hbm, vals2_hbm, dense_hbm, out_hbm,
             colsm, rowsm, valsm, gath, acc, semg, sems, semm):
        c = lax.axis_index("c")
        s = lax.axis_index("s")
        tile = c * 16 + s if edge_split else s
        row0 = tile * nrt
        cplane = 0 if edge_split else c

        def issue_meta(m, p):
            r = row0 + m * BLK
            pltpu.async_copy(cols3_hbm.at[cplane, pl.ds(r, BLK)],
                             colsm.at[p], semm)
            pltpu.async_copy(rows2_hbm.at[pl.ds(r, BLK)], rowsm.at[p], semm)
            pltpu.async_copy(vals2_hbm.at[pl.ds(r, BLK)], valsm.at[p], semm)

        def wait_meta(m, p):
            r = row0 + m * BLK
            pltpu.make_async_copy(cols3_hbm.at[cplane, pl.ds(r, BLK)],
                                  colsm.at[p], semm).wait()
            pltpu.make_async_copy(rows2_hbm.at[pl.ds(r, BLK)],
                                  rowsm.at[p], semm).wait()
            pltpu.make_async_copy(vals2_hbm.at[pl.ds(r, BLK)],
                                  valsm.at[p], semm).wait()

        def issue_gather(p, r, b):
            pltpu.async_copy(dense_hbm.at[colsm.at[p, r]], gath.at[b], semg)

        def wait_gather(p, r, b):
            pltpu.make_async_copy(dense_hbm.at[colsm.at[p, r]],
                                  gath.at[b], semg).wait()

        def issue_scatter(p, r, b):
            pltpu.async_copy(gath.at[b], acc.at[rowsm.at[p, r]], sems,
                             add=True)

        def wait_scatter(p, r, b):
            pltpu.make_async_copy(gath.at[b], acc.at[rowsm.at[p, r]],
                                  sems).wait()

        def scale(p, r, b):
            gbuf = gath.at[b]

            def tloop(t, tc):
                vals16 = valsm[p, r, pl.ds(t * 16, 16)]
                for e in range(16):
                    v = vals16[e]
                    row = t * 16 + e
                    for j in range(DC // 16):
                        sl = pl.ds(j * 16, 16)
                        gbuf[row, sl] = gbuf[row, sl] * v
                return tc

            lax.fori_loop(0, CH // 16, tloop, 0)

        issue_meta(0, 0)

        zero16 = jnp.zeros((16,), jnp.float32)
        zbuf = gath.at[0]

        def zrow(i, carry):
            for j in range(DC // 16):
                zbuf[i, pl.ds(j * 16, 16)] = zero16
            return carry

        lax.fori_loop(0, CH, zrow, 0)
        for q in range(stripe // 125):
            pltpu.sync_copy(zbuf.at[pl.ds(0, 125)],
                            acc.at[pl.ds(s * stripe + q * 125, 125)])
        wait_meta(0, 0)
        plsc.subcore_barrier()
        issue_gather(0, 0, 0)

        def body(mm, carry):
            for mb in range(2):
                P = mb
                m = mm * 2 + mb

                @pl.when(m + 1 < nblocks)
                def _():
                    issue_meta(m + 1, 1 - P)

                for b8 in range(BLK):
                    k = m * BLK + b8
                    b = b8 % 2
                    q = 1 - b
                    pq, rq = (1 - P, BLK - 1) if b8 == 0 else (P, b8 - 1)

                    @pl.when(k >= 1)
                    def _():
                        wait_scatter(pq, rq, q)

                    if b8 == BLK - 1:
                        @pl.when(m + 1 < nblocks)
                        def _():
                            wait_meta(m + 1, 1 - P)
                            issue_gather(1 - P, 0, q)
                    else:
                        issue_gather(P, b8 + 1, q)

                    wait_gather(P, b8, b)
                    scale(P, b8, b)
                    issue_scatter(P, b8, b)
            return carry

        lax.fori_loop(0, nblocks // 2, body, 0)
        wait_scatter(1, BLK - 1, (BLK - 1) % 2)
        plsc.subcore_barrier()

        w0 = s * 632

        @pl.when(s < 15)
        def _():
            pltpu.sync_copy(acc.at[pl.ds(w0, 632)],
                            out_hbm.at[pl.ds(c * N + w0, 632)])

        @pl.when(s == 15)
        def _():
            pltpu.sync_copy(acc.at[pl.ds(15 * 632, N - 15 * 632)],
                            out_hbm.at[pl.ds(c * N + 15 * 632, N - 15 * 632)])

    return spmm


_spmm_dsplit = _make_spmm(edge_split=False)
_spmm_esplit = _make_spmm(edge_split=True)


def _relu_w2_body(h0_ref, h1_ref, w_ref, o_ref):
    h0 = jnp.maximum(h0_ref[...], 0.0)
    h1 = jnp.maximum(h1_ref[...], 0.0)
    w = w_ref[...]
    o_ref[...] = (
        lax.dot_general(h0, w[:DC], (((1,), (0,)), ((), ())),
                        preferred_element_type=jnp.float32)
        + lax.dot_general(h1, w[DC:], (((1,), (0,)), ((), ())),
                          preferred_element_type=jnp.float32))


def _relu_w2(h_2, w2):
    bm = 2000
    return pl.pallas_call(
        _relu_w2_body,
        grid=(N // bm,),
        in_specs=[
            pl.BlockSpec((bm, DC), lambda i: (i, 0)),
            pl.BlockSpec((bm, DC), lambda i: (i + N // bm, 0)),
            pl.BlockSpec((H1, H2), lambda i: (0, 0)),
        ],
        out_specs=pl.BlockSpec((bm, H2), lambda i: (i, 0)),
        out_shape=jax.ShapeDtypeStruct((N, H2), jnp.float32),
    )(h_2, h_2, w2)


def _gram_body(a0_ref, a1_ref, b0_ref, b1_ref, o_ref):
    a = a0_ref[...] + a1_ref[...]
    b = b0_ref[...] + b1_ref[...]
    o_ref[...] = lax.dot_general(a, b, (((1,), (1,)), ((), ())),
                                 preferred_element_type=jnp.float32)


def _gram(p_2):
    bm = 200
    g = N // bm
    return pl.pallas_call(
        _gram_body,
        grid=(g,),
        in_specs=[
            pl.BlockSpec((bm, H2), lambda i: (i, 0)),
            pl.BlockSpec((bm, H2), lambda i: (i + g, 0)),
            pl.BlockSpec((N, H2), lambda i: (0, 0)),
            pl.BlockSpec((N, H2), lambda i: (1, 0)),
        ],
        out_specs=pl.BlockSpec((bm, N), lambda i: (i, 0)),
        out_shape=jax.ShapeDtypeStruct((N, N), jnp.float32),
    )(p_2, p_2, p_2, p_2)


def _prep_edges(rows, cols, vals, k_dim):
    pad = E_PAD - E
    pad_idx = jnp.arange(pad, dtype=jnp.int32)
    rows2 = jnp.concatenate([rows, pad_idx % N]).reshape(NROW, CH)
    cols_p = jnp.concatenate([cols, pad_idx % k_dim]).reshape(NROW, CH)
    vals2 = jnp.concatenate(
        [vals, jnp.zeros((pad,), jnp.float32)]).reshape(NROW, CH)
    cols3 = jnp.stack([cols_p, cols_p + k_dim])
    return cols3, rows2, vals2


@jax.jit
def _run(feat_rows, feat_cols, feat_vals, adj_rows, adj_cols, adj_vals, W1, W2):
    fc3, fr2, fv2 = _prep_edges(feat_rows, feat_cols, feat_vals, F)
    ac3, ar2, av2 = _prep_edges(adj_rows, adj_cols, adj_vals, N)
    w1_2 = jnp.concatenate([W1[:, :DC], W1[:, DC:]], axis=0)
    h1_2 = _spmm_dsplit(fc3, fr2, fv2, w1_2)
    h_2 = _spmm_dsplit(ac3, ar2, av2, h1_2)
    h2 = _relu_w2(h_2, W2)
    p_2 = _spmm_esplit(ac3, ar2, av2, h2)
    recon = _gram(p_2)
    return recon.astype(jnp.float64)


def kernel(feat_rows, feat_cols, feat_vals, adj_rows, adj_cols, adj_vals, W1, W2):
    return _run(feat_rows, feat_cols, feat_vals, adj_rows, adj_cols, adj_vals, W1, W2)


# speedup vs baseline: 2.4441x; 1.0011x over previous
import functools

import jax
import jax.numpy as jnp
from jax import lax
from jax.experimental import pallas as pl
from jax.experimental.pallas import tpu as pltpu
from jax.experimental.pallas import tpu_sc as plsc

N = 10000
F = 512
H1 = 256
H2 = 128
E = 320000
CH = 128
DC = 128
E_PAD = 327680
NROW = E_PAD // CH


def _make_spmm(edge_split, dense_rows_spmem=0):
    n_tiles = 32 if edge_split else 16
    nrt = NROW // n_tiles
    BLK = 4
    nblocks = nrt // BLK
    stripe = N // 16

    mesh = plsc.VectorSubcoreMesh(core_axis_name="c", subcore_axis_name="s")

    spmem_dense_scratch = (
        [pltpu.VMEM_SHARED((dense_rows_spmem, DC), jnp.float32)]
        if dense_rows_spmem else [])

    @functools.partial(
        pl.kernel,
        out_type=jax.ShapeDtypeStruct((2 * N, DC), jnp.float32),
        mesh=mesh,
        scratch_types=[
            pltpu.VMEM((2, BLK, CH), jnp.int32),
            pltpu.VMEM((2, BLK, CH), jnp.int32),
            pltpu.VMEM((2, BLK, CH), jnp.float32),
            pltpu.VMEM((2, CH, DC), jnp.float32),
            pltpu.VMEM_SHARED((N, DC), jnp.float32),
            pltpu.SemaphoreType.DMA,
            pltpu.SemaphoreType.DMA,
            pltpu.SemaphoreType.DMA,
        ] + spmem_dense_scratch,
    )
    def spmm(cols3_hbm, rows2_hbm, vals2_hbm, dense_hbm, out_hbm,
             colsm, rowsm, valsm, gath, acc, semg, sems, semm,
             *maybe_wdense):
        c = lax.axis_index("c")
        s = lax.axis_index("s")
        tile = c * 16 + s if edge_split else s
        row0 = tile * nrt
        cplane = 0 if (edge_split or dense_rows_spmem) else c
        if dense_rows_spmem:
            dense_src = maybe_wdense[0]
        else:
            dense_src = dense_hbm

        def issue_meta(m, p):
            r = row0 + m * BLK
            pltpu.async_copy(cols3_hbm.at[cplane, pl.ds(r, BLK)],
                             colsm.at[p], semm)
            pltpu.async_copy(rows2_hbm.at[pl.ds(r, BLK)], rowsm.at[p], semm)
            pltpu.async_copy(vals2_hbm.at[pl.ds(r, BLK)], valsm.at[p], semm)

        def wait_meta(m, p):
            r = row0 + m * BLK
            pltpu.make_async_copy(cols3_hbm.at[cplane, pl.ds(r, BLK)],
                                  colsm.at[p], semm).wait()
            pltpu.make_async_copy(rows2_hbm.at[pl.ds(r, BLK)],
                                  rowsm.at[p], semm).wait()
            pltpu.make_async_copy(vals2_hbm.at[pl.ds(r, BLK)],
                                  valsm.at[p], semm).wait()

        def issue_gather(p, r, b):
            pltpu.async_copy(dense_src.at[colsm.at[p, r]], gath.at[b], semg)

        def wait_gather(p, r, b):
            pltpu.make_async_copy(dense_src.at[colsm.at[p, r]],
                                  gath.at[b], semg).wait()

        def issue_scatter(p, r, b):
            pltpu.async_copy(gath.at[b], acc.at[rowsm.at[p, r]], sems,
                             add=True)

        def wait_scatter(p, r, b):
            pltpu.make_async_copy(gath.at[b], acc.at[rowsm.at[p, r]],
                                  sems).wait()

        def scale(p, r, b):
            gbuf = gath.at[b]

            def tloop(t, tc):
                vals16 = valsm[p, r, pl.ds(t * 16, 16)]
                for e in range(16):
                    v = vals16[e]
                    row = t * 16 + e
                    for j in range(DC // 16):
                        sl = pl.ds(j * 16, 16)
                        gbuf[row, sl] = gbuf[row, sl] * v
                return tc

            lax.fori_loop(0, CH // 16, tloop, 0)

        issue_meta(0, 0)
        if dense_rows_spmem:
            drt = dense_rows_spmem // 16
            pltpu.sync_copy(
                dense_hbm.at[pl.ds(c * dense_rows_spmem + s * drt, drt)],
                maybe_wdense[0].at[pl.ds(s * drt, drt)])

        zero16 = jnp.zeros((16,), jnp.float32)
        zbuf = gath.at[0]

        def zrow(i, carry):
            for j in range(DC // 16):
                zbuf[i, pl.ds(j * 16, 16)] = zero16
            return carry

        lax.fori_loop(0, CH, zrow, 0)
        for q in range(stripe // 125):
            pltpu.sync_copy(zbuf.at[pl.ds(0, 125)],
                            acc.at[pl.ds(s * stripe + q * 125, 125)])
        wait_meta(0, 0)
        plsc.subcore_barrier()
        issue_gather(0, 0, 0)

        def body(mm, carry):
            for mb in range(2):
                P = mb
                m = mm * 2 + mb

                @pl.when(m + 1 < nblocks)
                def _():
                    issue_meta(m + 1, 1 - P)

                for b8 in range(BLK):
                    k = m * BLK + b8
                    b = b8 % 2
                    q = 1 - b
                    pq, rq = (1 - P, BLK - 1) if b8 == 0 else (P, b8 - 1)

                    @pl.when(k >= 1)
                    def _():
                        wait_scatter(pq, rq, q)

                    if b8 == BLK - 1:
                        @pl.when(m + 1 < nblocks)
                        def _():
                            wait_meta(m + 1, 1 - P)
                            issue_gather(1 - P, 0, q)
                    else:
                        issue_gather(P, b8 + 1, q)

                    wait_gather(P, b8, b)
                    scale(P, b8, b)
                    issue_scatter(P, b8, b)
            return carry

        lax.fori_loop(0, nblocks // 2, body, 0)
        wait_scatter(1, BLK - 1, (BLK - 1) % 2)
        plsc.subcore_barrier()

        w0 = s * 632

        @pl.when(s < 15)
        def _():
            pltpu.sync_copy(acc.at[pl.ds(w0, 632)],
                            out_hbm.at[pl.ds(c * N + w0, 632)])

        @pl.when(s == 15)
        def _():
            pltpu.sync_copy(acc.at[pl.ds(15 * 632, N - 15 * 632)],
                            out_hbm.at[pl.ds(c * N + 15 * 632, N - 15 * 632)])

    return spmm


_spmm_dsplit = _make_spmm(edge_split=False)
_spmm_esplit = _make_spmm(edge_split=True)
_spmm_feat = _make_spmm(edge_split=False, dense_rows_spmem=F)


def _relu_w2_body(h0_ref, h1_ref, w_ref, o_ref):
    h0 = jnp.maximum(h0_ref[...], 0.0)
    h1 = jnp.maximum(h1_ref[...], 0.0)
    w = w_ref[...]
    o_ref[...] = (
        lax.dot_general(h0, w[:DC], (((1,), (0,)), ((), ())),
                        preferred_element_type=jnp.float32)
        + lax.dot_general(h1, w[DC:], (((1,), (0,)), ((), ())),
                          preferred_element_type=jnp.float32))


def _relu_w2(h_2, w2):
    bm = 2000
    return pl.pallas_call(
        _relu_w2_body,
        grid=(N // bm,),
        in_specs=[
            pl.BlockSpec((bm, DC), lambda i: (i, 0)),
            pl.BlockSpec((bm, DC), lambda i: (i + N // bm, 0)),
            pl.BlockSpec((H1, H2), lambda i: (0, 0)),
        ],
        out_specs=pl.BlockSpec((bm, H2), lambda i: (i, 0)),
        out_shape=jax.ShapeDtypeStruct((N, H2), jnp.float32),
    )(h_2, h_2, w2)


def _gram_body(a0_ref, a1_ref, b0_ref, b1_ref, o_ref):
    a = a0_ref[...] + a1_ref[...]
    b = b0_ref[...] + b1_ref[...]
    o_ref[...] = lax.dot_general(a, b, (((1,), (1,)), ((), ())),
                                 preferred_element_type=jnp.float32)


def _gram(p_2):
    bm = 200
    g = N // bm
    return pl.pallas_call(
        _gram_body,
        grid=(g,),
        in_specs=[
            pl.BlockSpec((bm, H2), lambda i: (i, 0)),
            pl.BlockSpec((bm, H2), lambda i: (i + g, 0)),
            pl.BlockSpec((N, H2), lambda i: (0, 0)),
            pl.BlockSpec((N, H2), lambda i: (1, 0)),
        ],
        out_specs=pl.BlockSpec((bm, N), lambda i: (i, 0)),
        out_shape=jax.ShapeDtypeStruct((N, N), jnp.float32),
    )(p_2, p_2, p_2, p_2)


def _prep_edges(rows, cols, vals, k_dim):
    pad = E_PAD - E
    pad_idx = jnp.arange(pad, dtype=jnp.int32)
    rows2 = jnp.concatenate([rows, pad_idx % N]).reshape(NROW, CH)
    cols_p = jnp.concatenate([cols, pad_idx % k_dim]).reshape(NROW, CH)
    vals2 = jnp.concatenate(
        [vals, jnp.zeros((pad,), jnp.float32)]).reshape(NROW, CH)
    cols3 = jnp.stack([cols_p, cols_p + k_dim])
    return cols3, rows2, vals2


@jax.jit
def _run(feat_rows, feat_cols, feat_vals, adj_rows, adj_cols, adj_vals, W1, W2):
    fc3, fr2, fv2 = _prep_edges(feat_rows, feat_cols, feat_vals, F)
    ac3, ar2, av2 = _prep_edges(adj_rows, adj_cols, adj_vals, N)
    w1_2 = jnp.concatenate([W1[:, :DC], W1[:, DC:]], axis=0)
    h1_2 = _spmm_feat(fc3, fr2, fv2, w1_2)
    h_2 = _spmm_dsplit(ac3, ar2, av2, h1_2)
    h2 = _relu_w2(h_2, W2)
    p_2 = _spmm_esplit(ac3, ar2, av2, h2)
    recon = _gram(p_2)
    return recon.astype(jnp.float64)


def kernel(feat_rows, feat_cols, feat_vals, adj_rows, adj_cols, adj_vals, W1, W2):
    return _run(feat_rows, feat_cols, feat_vals, adj_rows, adj_cols, adj_vals, W1, W2)
